# baseline (speedup 1.0000x reference)
import jax
import jax.numpy as jnp
from jax import lax
from jax.experimental import pallas as pl
from jax.experimental.pallas import tpu as pltpu
from jax.experimental.pallas import tpu_sc as plsc

B = 16384
V = 1000000
D = 32
CTX = 20
NEG = 20

NC = 2
NS = 16
NW = NC * NS
RPW = B // NW
CH = 128
NCHUNK = RPW // CH
IPC = CH * CTX
KROWS = IPC // 128

NTILES = V // 128
TPW = NTILES // NW
TREM = NTILES - TPW * NW


def _relay_body(tabT_i, tabT_o, tail_i, tail_o, lin_i, lin_o, buf, obuf, sem):
    wid = lax.axis_index("s") * NC + lax.axis_index("c")
    base = wid * TPW + jnp.minimum(wid, TREM)
    cnt = TPW + jnp.where(wid < TREM, 1, 0)
    pat0 = lax.iota(jnp.int32, 16) // 8
    pat1 = lax.iota(jnp.int32, 16) % 8

    for tabT, lin in ((tabT_i, lin_i), (tabT_o, lin_o)):
        def tile_body(t, carry, tabT=tabT, lin=lin):
            c = base + t
            for r in range(4):
                pltpu.async_copy(
                    tabT.at[pl.ds(r * 8, 8), pl.ds(c * 128, 128)],
                    buf.at[r], sem)
            for r in range(4):
                pltpu.make_async_copy(
                    tabT.at[pl.ds(r * 8, 8), pl.ds(c * 128, 128)],
                    buf.at[r], sem).wait()

            def v_body(vb, cc):
                for k in range(4):
                    vcol = jnp.full((16,), vb * 4 + k, jnp.int32)
                    g0 = plsc.load_gather(buf, [pat0, pat1, vcol])
                    g1 = plsc.load_gather(buf, [pat0 + 2, pat1, vcol])
                    obuf[vb, pl.ds(k * 32, 16)] = g0
                    obuf[vb, pl.ds(k * 32 + 16, 16)] = g1
                return cc
            lax.fori_loop(0, 32, v_body, 0)
            pltpu.sync_copy(obuf, lin.at[pl.ds(c * 32, 32)])
            return carry
        lax.fori_loop(0, cnt, tile_body, 0)

    @pl.when(wid == 0)
    def _tail():
        for tail, lin in ((tail_i, lin_i), (tail_o, lin_o)):
            for r in range(4):
                pltpu.sync_copy(tail.at[pl.ds(r * 8, 8)], buf.at[r])

            def v_body(vb, cc):
                for k in range(4):
                    vcol = jnp.full((16,), vb * 4 + k, jnp.int32)
                    g0 = plsc.load_gather(buf, [pat0, pat1, vcol])
                    g1 = plsc.load_gather(buf, [pat0 + 2, pat1, vcol])
                    obuf[vb, pl.ds(k * 32, 16)] = g0
                    obuf[vb, pl.ds(k * 32 + 16, 16)] = g1
                return cc
            lax.fori_loop(0, 16, v_body, 0)
            pltpu.sync_copy(obuf.at[pl.ds(0, 16)],
                            lin.at[pl.ds(NTILES * 32, 16)])


_relay_call = pl.kernel(
    _relay_body,
    out_type=[jax.ShapeDtypeStruct((V // 4, 128), jnp.float32),
              jax.ShapeDtypeStruct((V // 4, 128), jnp.float32)],
    mesh=plsc.VectorSubcoreMesh(core_axis_name="c", subcore_axis_name="s"),
    compiler_params=pltpu.CompilerParams(needs_layout_passes=False,
                                         use_tc_tiling_on_sc=True),
    scratch_types=[
        pltpu.VMEM((4, 8, 128), jnp.float32),
        pltpu.VMEM((32, 128), jnp.float32),
        pltpu.SemaphoreType.DMA,
    ],
)


def _sc_body(inp_embed, out_embed, inp2d, out2d, neg2d,
             pos_hbm, negT_hbm,
             idxa, idxb, idxo, rows_v, out_rows, ctx_v, pos_v, negT_v,
             sem, sem2):
    wid = lax.axis_index("s") * NC + lax.axis_index("c")
    iota = lax.iota(jnp.int32, 16)

    def chunk_body(c, carry):
        ioff = (wid * RPW + c * CH) * CTX
        pltpu.sync_copy(inp2d.at[pl.ds(ioff, IPC)], idxa)
        handles = []
        for k in range(KROWS):
            handles.append(pltpu.async_copy(
                inp_embed.at[idxa.at[pl.ds(k * 128, 128)]],
                rows_v.at[pl.ds(k * 128, 128)], sem))
        pltpu.sync_copy(out2d.at[pl.ds(wid * RPW + c * CH, CH)], idxo)
        oh = pltpu.async_copy(out_embed.at[idxo], out_rows, sem2)
        for h in handles:
            h.wait()

        def ctx_sum_body(i, cc):
            m0 = i * CTX
            a0 = rows_v[m0, pl.ds(0, 16)]
            a1 = rows_v[m0, pl.ds(16, 16)]
            for k in range(1, CTX):
                a0 = a0 + rows_v[m0 + k, pl.ds(0, 16)]
                a1 = a1 + rows_v[m0 + k, pl.ds(16, 16)]
            ctx_v[i, pl.ds(0, 16)] = a0
            ctx_v[i, pl.ds(16, 16)] = a1
            return cc
        lax.fori_loop(0, CH, ctx_sum_body, 0)

        pltpu.sync_copy(neg2d.at[pl.ds(ioff, IPC)], idxb)
        nh = []
        for k in range(KROWS):
            nh.append(pltpu.async_copy(
                out_embed.at[idxb.at[pl.ds(k * 128, 128)]],
                rows_v.at[pl.ds(k * 128, 128)], sem))
        oh.wait()
        for h in nh:
            h.wait()

        def group_body(g, cc):
            rows16 = g * 16 + iota
            negrows = [rows16 * NEG + j for j in range(NEG)]

            def d_body(dcol, acc):
                col = jnp.full((16,), dcol, jnp.int32)
                cd = plsc.load_gather(ctx_v, [rows16, col])
                od = plsc.load_gather(out_rows, [rows16, col])
                pos = acc[0] + cd * od
                new = tuple(
                    acc[1 + j] + plsc.load_gather(rows_v, [negrows[j], col]) * cd
                    for j in range(NEG))
                return (pos,) + new

            init = tuple(jnp.zeros((16,), jnp.float32) for _ in range(NEG + 1))
            res = lax.fori_loop(0, D, d_body, init)
            off = c * CH + g * 16
            pos_v[pl.ds(off, 16)] = res[0]
            for j in range(NEG):
                negT_v[j, pl.ds(off, 16)] = res[1 + j]
            return cc
        lax.fori_loop(0, CH // 16, group_body, 0)
        return carry

    lax.fori_loop(0, NCHUNK, chunk_body, 0)
    pltpu.sync_copy(pos_v, pos_hbm.at[pl.ds(wid * RPW, RPW)])
    pltpu.sync_copy(negT_v, negT_hbm.at[:, pl.ds(wid * RPW, RPW)])


_sc_call = pl.kernel(
    _sc_body,
    out_type=[jax.ShapeDtypeStruct((B,), jnp.float32),
              jax.ShapeDtypeStruct((NEG, B), jnp.float32)],
    mesh=plsc.VectorSubcoreMesh(core_axis_name="c", subcore_axis_name="s"),
    compiler_params=pltpu.CompilerParams(needs_layout_passes=False,
                                         use_tc_tiling_on_sc=False),
    scratch_types=[
        pltpu.VMEM((IPC,), jnp.int32),
        pltpu.VMEM((IPC,), jnp.int32),
        pltpu.VMEM((CH,), jnp.int32),
        pltpu.VMEM((IPC, D), jnp.float32),
        pltpu.VMEM((CH, D), jnp.float32),
        pltpu.VMEM((CH, D), jnp.float32),
        pltpu.VMEM((RPW,), jnp.float32),
        pltpu.VMEM((NEG, RPW), jnp.float32),
        pltpu.SemaphoreType.DMA,
        pltpu.SemaphoreType.DMA,
    ],
)


def _log_sigmoid(x):
    return jnp.minimum(x, 0.0) - jnp.log(1.0 + jnp.exp(-jnp.abs(x)))


def _tc_body(pos_ref, neg_ref, o_ref):
    pos = pos_ref[...] * (1.0 / CTX)
    neg = neg_ref[...] * (1.0 / CTX)
    t1 = jnp.mean(_log_sigmoid(pos))
    t2 = jnp.sum(_log_sigmoid(-neg)) * (1.0 / B)
    o_ref[0, 0] = -(t1 + t2)


_tc_call = pl.pallas_call(
    _tc_body,
    out_shape=jax.ShapeDtypeStruct((1, 1), jnp.float32),
    out_specs=pl.BlockSpec(memory_space=pltpu.SMEM),
)


def kernel(inp_embed, out_embed, inp, out, neg):
    inp_i = inp.astype(jnp.int32).reshape(B * CTX)
    out_i = out.astype(jnp.int32).reshape(B)
    neg_i = neg.astype(jnp.int32).reshape(B * NEG)
    pad = ((0, 0), (0, 64))
    tail_i = jnp.pad(lax.slice(inp_embed.T, (0, NTILES * 128), (D, V)), pad)
    tail_o = jnp.pad(lax.slice(out_embed.T, (0, NTILES * 128), (D, V)), pad)
    lin_i, lin_o = _relay_call(inp_embed.T, out_embed.T, tail_i, tail_o)
    pos, negT = _sc_call(lin_i.reshape(V, D), lin_o.reshape(V, D),
                         inp_i, out_i, neg_i)
    loss = _tc_call(pos.reshape(128, 128), negT.reshape(NEG * B // 128, 128))
    return loss[0, 0]


# speedup vs baseline: 2.5852x; 2.5852x over previous
import jax
import jax.numpy as jnp
from jax import lax
from jax.experimental import pallas as pl
from jax.experimental.pallas import tpu as pltpu
from jax.experimental.pallas import tpu_sc as plsc

B = 16384
V = 1000000
D = 32
CTX = 20
NEG = 20

NC = 2
NS = 16
NW = NC * NS
RPW = B // NW
CH = 64
NCHUNK = RPW // CH
IPC = CH * CTX
KROWS = IPC // 128

NTILES = V // 128
TPW = NTILES // NW
TREM = NTILES - TPW * NW


def _transpose_tile(bslot, oslot, iota, i0s, i1base, colvs):
    for d0 in range(32):
        dvec = lax.rem(d0 + iota, 32)
        dblkv = dvec // 8
        dsubv = lax.rem(dvec, 8)
        i1v = i1base + dvec
        vals = [plsc.load_gather(bslot, [dblkv, dsubv, colvs[g]])
                for g in range(8)]
        for ot in range(4):
            for vgl in range(2):
                plsc.store_scatter(oslot.at[ot], [i0s[vgl], i1v],
                                   vals[ot * 2 + vgl])


def _relay_body(tabT_i, tabT_o, tail_i, tail_o, lin_i, lin_o, buf, obuf,
                sem, sem2):
    wid = lax.axis_index("s") * NC + lax.axis_index("c")
    base = wid * TPW
    iota = lax.iota(jnp.int32, 16)
    i0s = (iota // 4, iota // 4 + 4)
    i1base = (iota % 4) * 32
    colvs = tuple(iota + 16 * g for g in range(8))

    def fire_in(tabT, c, s):
        for r in range(4):
            pltpu.async_copy(
                tabT.at[pl.ds(r * 8, 8), pl.ds(c * 128, 128)],
                buf.at[s, r], sem.at[s])

    def wait_in(tabT, c, s):
        for r in range(4):
            pltpu.make_async_copy(
                tabT.at[pl.ds(r * 8, 8), pl.ds(c * 128, 128)],
                buf.at[s, r], sem.at[s]).wait()

    def out_dma(lin, c, s, fire):
        for r in range(4):
            cp = pltpu.make_async_copy(
                obuf.at[s, r], lin.at[pl.ds(c * 32 + r * 8, 8)],
                sem2.at[s])
            cp.start() if fire else cp.wait()

    for tabT, lin in ((tabT_i, lin_i), (tabT_o, lin_o)):
        for s in range(2):
            fire_in(tabT, base + s, s)

        def pair_body(q, carry, tabT=tabT, lin=lin):
            for s in range(2):
                t = base + 2 * q + s

                @pl.when(q >= 1)
                def _wait_out(t=t, s=s):
                    out_dma(lin, t - 2, s, fire=False)

                @pl.when(2 * q + s + 2 < TPW)
                def _fire_next(t=t, s=s):
                    fire_in(tabT, t + 2, s)
                wait_in(tabT, t, s)
                _transpose_tile(buf.at[s], obuf.at[s], iota, i0s, i1base,
                                colvs)
                out_dma(lin, t, s, fire=True)
            return carry
        lax.fori_loop(0, TPW // 2, pair_body, 0)
        for s in range(2):
            out_dma(lin, base + TPW - 2 + s, s, fire=False)

        @pl.when(wid < NTILES - NW * TPW)
        def _leftover(tabT=tabT, lin=lin):
            c = NW * TPW + wid
            fire_in(tabT, c, 0)
            wait_in(tabT, c, 0)
            _transpose_tile(buf.at[0], obuf.at[0], iota, i0s, i1base, colvs)
            out_dma(lin, c, 0, fire=True)
            out_dma(lin, c, 0, fire=False)

    @pl.when(wid == 0)
    def _tail():
        for tail, lin in ((tail_i, lin_i), (tail_o, lin_o)):
            for r in range(4):
                pltpu.sync_copy(tail.at[pl.ds(r * 8, 8)], buf.at[0, r])
            _transpose_tile(buf.at[0], obuf.at[0], iota, i0s, i1base, colvs)
            for r in range(2):
                pltpu.sync_copy(obuf.at[0, r],
                                lin.at[pl.ds(NTILES * 32 + r * 8, 8)])


_relay_call = pl.kernel(
    _relay_body,
    out_type=[jax.ShapeDtypeStruct((V // 4, 128), jnp.float32),
              jax.ShapeDtypeStruct((V // 4, 128), jnp.float32)],
    mesh=plsc.VectorSubcoreMesh(core_axis_name="c", subcore_axis_name="s"),
    compiler_params=pltpu.CompilerParams(needs_layout_passes=False,
                                         use_tc_tiling_on_sc=True),
    scratch_types=[
        pltpu.VMEM((2, 4, 8, 128), jnp.float32),
        pltpu.VMEM((2, 4, 8, 128), jnp.float32),
        pltpu.SemaphoreType.DMA((2,)),
        pltpu.SemaphoreType.DMA((2,)),
    ],
)


def _sc_body(inp_embed, out_embed, inp2d, out2d, neg2d,
             pos_hbm, negT_hbm,
             idxa, idxb, idxo, rows_a, rows_b, out_rows, ctx_v, pos_v,
             negT_v, sem, sem2):
    wid = lax.axis_index("s") * NC + lax.axis_index("c")
    iota = lax.iota(jnp.int32, 16)

    def fire_ctx(c):
        pltpu.sync_copy(inp2d.at[pl.ds((wid * RPW + c * CH) * CTX, IPC)],
                        idxa)
        for k in range(KROWS):
            pltpu.async_copy(
                inp_embed.at[idxa.at[pl.ds(k * 128, 128)]],
                rows_a.at[pl.ds(k * 128, 128)], sem)

    def wait_ctx():
        for k in range(KROWS):
            pltpu.make_async_copy(
                inp_embed.at[idxa.at[pl.ds(k * 128, 128)]],
                rows_a.at[pl.ds(k * 128, 128)], sem).wait()

    fire_ctx(0)

    def chunk_body(c, carry):
        pltpu.sync_copy(neg2d.at[pl.ds((wid * RPW + c * CH) * CTX, IPC)],
                        idxb)
        pltpu.sync_copy(out2d.at[pl.ds(wid * RPW + c * CH, CH)], idxo)
        nh = []
        for k in range(KROWS):
            nh.append(pltpu.async_copy(
                out_embed.at[idxb.at[pl.ds(k * 128, 128)]],
                rows_b.at[pl.ds(k * 128, 128)], sem2))
        oh = pltpu.async_copy(out_embed.at[idxo], out_rows, sem2)

        wait_ctx()

        def ctx_sum_body(i, cc):
            m0 = i * CTX
            a0 = rows_a[m0, pl.ds(0, 16)]
            a1 = rows_a[m0, pl.ds(16, 16)]
            for k in range(1, CTX):
                a0 = a0 + rows_a[m0 + k, pl.ds(0, 16)]
                a1 = a1 + rows_a[m0 + k, pl.ds(16, 16)]
            ctx_v[i, pl.ds(0, 16)] = a0
            ctx_v[i, pl.ds(16, 16)] = a1
            return cc
        lax.fori_loop(0, CH, ctx_sum_body, 0)

        @pl.when(c + 1 < NCHUNK)
        def _prefetch():
            fire_ctx(c + 1)

        for h in nh:
            h.wait()
        oh.wait()

        def group_body(g, cc):
            rows16 = g * 16 + iota
            negrows = [rows16 * NEG + j for j in range(NEG)]

            def d_body(dcol, acc):
                col = jnp.full((16,), dcol, jnp.int32)
                cd = plsc.load_gather(ctx_v, [rows16, col])
                od = plsc.load_gather(out_rows, [rows16, col])
                pos = acc[0] + cd * od
                new = tuple(
                    acc[1 + j] + plsc.load_gather(rows_b, [negrows[j], col]) * cd
                    for j in range(NEG))
                return (pos,) + new

            init = tuple(jnp.zeros((16,), jnp.float32) for _ in range(NEG + 1))
            res = lax.fori_loop(0, D, d_body, init)
            off = c * CH + g * 16
            pos_v[pl.ds(off, 16)] = res[0]
            for j in range(NEG):
                negT_v[j, pl.ds(off, 16)] = res[1 + j]
            return cc
        lax.fori_loop(0, CH // 16, group_body, 0)
        return carry

    lax.fori_loop(0, NCHUNK, chunk_body, 0)
    pltpu.sync_copy(pos_v, pos_hbm.at[pl.ds(wid * RPW, RPW)])
    pltpu.sync_copy(negT_v, negT_hbm.at[:, pl.ds(wid * RPW, RPW)])


_sc_call = pl.kernel(
    _sc_body,
    out_type=[jax.ShapeDtypeStruct((B,), jnp.float32),
              jax.ShapeDtypeStruct((NEG, B), jnp.float32)],
    mesh=plsc.VectorSubcoreMesh(core_axis_name="c", subcore_axis_name="s"),
    compiler_params=pltpu.CompilerParams(needs_layout_passes=False,
                                         use_tc_tiling_on_sc=False),
    scratch_types=[
        pltpu.VMEM((IPC,), jnp.int32),
        pltpu.VMEM((IPC,), jnp.int32),
        pltpu.VMEM((CH,), jnp.int32),
        pltpu.VMEM((IPC, D), jnp.float32),
        pltpu.VMEM((IPC, D), jnp.float32),
        pltpu.VMEM((CH, D), jnp.float32),
        pltpu.VMEM((CH, D), jnp.float32),
        pltpu.VMEM((RPW,), jnp.float32),
        pltpu.VMEM((NEG, RPW), jnp.float32),
        pltpu.SemaphoreType.DMA,
        pltpu.SemaphoreType.DMA,
    ],
)


def _log_sigmoid(x):
    return jnp.minimum(x, 0.0) - jnp.log(1.0 + jnp.exp(-jnp.abs(x)))


def _tc_body(pos_ref, neg_ref, o_ref):
    pos = pos_ref[...] * (1.0 / CTX)
    neg = neg_ref[...] * (1.0 / CTX)
    t1 = jnp.mean(_log_sigmoid(pos))
    t2 = jnp.sum(_log_sigmoid(-neg)) * (1.0 / B)
    o_ref[0, 0] = -(t1 + t2)


_tc_call = pl.pallas_call(
    _tc_body,
    out_shape=jax.ShapeDtypeStruct((1, 1), jnp.float32),
    out_specs=pl.BlockSpec(memory_space=pltpu.SMEM),
)


def kernel(inp_embed, out_embed, inp, out, neg):
    inp_i = inp.astype(jnp.int32).reshape(B * CTX)
    out_i = out.astype(jnp.int32).reshape(B)
    neg_i = neg.astype(jnp.int32).reshape(B * NEG)
    pad = ((0, 0), (0, 64))
    tail_i = jnp.pad(lax.slice(inp_embed.T, (0, NTILES * 128), (D, V)), pad)
    tail_o = jnp.pad(lax.slice(out_embed.T, (0, NTILES * 128), (D, V)), pad)
    lin_i, lin_o = _relay_call(inp_embed.T, out_embed.T, tail_i, tail_o)
    pos, negT = _sc_call(lin_i.reshape(V, D), lin_o.reshape(V, D),
                         inp_i, out_i, neg_i)
    loss = _tc_call(pos.reshape(128, 128), negT.reshape(NEG * B // 128, 128))
    return loss[0, 0]


# speedup vs baseline: 2.6541x; 1.0266x over previous
import jax
import jax.numpy as jnp
from jax import lax
from jax.experimental import pallas as pl
from jax.experimental.pallas import tpu as pltpu
from jax.experimental.pallas import tpu_sc as plsc

B = 16384
V = 1000000
D = 32
CTX = 20
NEG = 20

NC = 2
NS = 16
NW = NC * NS
RPW = B // NW
CH = 64
NCHUNK = RPW // CH
IPC = CH * CTX
KROWS = IPC // 128

NTILES = V // 128
TPW = NTILES // NW
TREM = NTILES - TPW * NW


def _transpose_tile(bslot, oslot, iota, i0s, i1base, colvs):
    for d0 in range(32):
        dvec = lax.rem(d0 + iota, 32)
        dblkv = dvec // 8
        dsubv = lax.rem(dvec, 8)
        i1v = i1base + dvec
        vals = [plsc.load_gather(bslot, [dblkv, dsubv, colvs[g]])
                for g in range(8)]
        for ot in range(4):
            for vgl in range(2):
                plsc.store_scatter(oslot.at[ot], [i0s[vgl], i1v],
                                   vals[ot * 2 + vgl])


def _relay_body(tabT_i, tabT_o, tail_i, tail_o, lin_i, lin_o, buf, obuf,
                sem, sem2):
    wid = lax.axis_index("s") * NC + lax.axis_index("c")
    base = wid * TPW
    iota = lax.iota(jnp.int32, 16)
    i0s = (iota // 4, iota // 4 + 4)
    i1base = (iota % 4) * 32
    colvs = tuple(iota + 16 * g for g in range(8))

    def fire_in(tabT, c, s):
        for r in range(4):
            pltpu.async_copy(
                tabT.at[pl.ds(r * 8, 8), pl.ds(c * 128, 128)],
                buf.at[s, r], sem.at[s])

    def wait_in(tabT, c, s):
        for r in range(4):
            pltpu.make_async_copy(
                tabT.at[pl.ds(r * 8, 8), pl.ds(c * 128, 128)],
                buf.at[s, r], sem.at[s]).wait()

    def out_dma(lin, c, s, fire):
        for r in range(4):
            cp = pltpu.make_async_copy(
                obuf.at[s, r], lin.at[pl.ds(c * 32 + r * 8, 8)],
                sem2.at[s])
            cp.start() if fire else cp.wait()

    for tabT, lin in ((tabT_i, lin_i), (tabT_o, lin_o)):
        for s in range(2):
            fire_in(tabT, base + s, s)

        def pair_body(q, carry, tabT=tabT, lin=lin):
            for s in range(2):
                t = base + 2 * q + s

                @pl.when(q >= 1)
                def _wait_out(t=t, s=s):
                    out_dma(lin, t - 2, s, fire=False)

                wait_in(tabT, t, s)
                _transpose_tile(buf.at[s], obuf.at[s], iota, i0s, i1base,
                                colvs)

                @pl.when(2 * q + s + 2 < TPW)
                def _fire_next(t=t, s=s):
                    fire_in(tabT, t + 2, s)
                out_dma(lin, t, s, fire=True)
            return carry
        lax.fori_loop(0, TPW // 2, pair_body, 0)
        for s in range(2):
            out_dma(lin, base + TPW - 2 + s, s, fire=False)

        @pl.when(wid < NTILES - NW * TPW)
        def _leftover(tabT=tabT, lin=lin):
            c = NW * TPW + wid
            fire_in(tabT, c, 0)
            wait_in(tabT, c, 0)
            _transpose_tile(buf.at[0], obuf.at[0], iota, i0s, i1base, colvs)
            out_dma(lin, c, 0, fire=True)
            out_dma(lin, c, 0, fire=False)

    @pl.when(wid == 0)
    def _tail():
        for tail, lin in ((tail_i, lin_i), (tail_o, lin_o)):
            for r in range(4):
                pltpu.sync_copy(tail.at[pl.ds(r * 8, 8)], buf.at[0, r])
            _transpose_tile(buf.at[0], obuf.at[0], iota, i0s, i1base, colvs)
            for r in range(2):
                pltpu.sync_copy(obuf.at[0, r],
                                lin.at[pl.ds(NTILES * 32 + r * 8, 8)])


_relay_call = pl.kernel(
    _relay_body,
    out_type=[jax.ShapeDtypeStruct((V // 4, 128), jnp.float32),
              jax.ShapeDtypeStruct((V // 4, 128), jnp.float32)],
    mesh=plsc.VectorSubcoreMesh(core_axis_name="c", subcore_axis_name="s"),
    compiler_params=pltpu.CompilerParams(needs_layout_passes=False,
                                         use_tc_tiling_on_sc=True),
    scratch_types=[
        pltpu.VMEM((2, 4, 8, 128), jnp.float32),
        pltpu.VMEM((2, 4, 8, 128), jnp.float32),
        pltpu.SemaphoreType.DMA((2,)),
        pltpu.SemaphoreType.DMA((2,)),
    ],
)


def _sc_body(inp_embed, out_embed, inp2d, out2d, neg2d,
             pos_hbm, negT_hbm,
             idxa, idxb, idxo, rows_a, rows_b, out_rows, ctx_v, pos_v,
             negT_v, sem, sem2):
    wid = lax.axis_index("s") * NC + lax.axis_index("c")
    iota = lax.iota(jnp.int32, 16)

    def fire_ctx(c):
        pltpu.sync_copy(inp2d.at[pl.ds((wid * RPW + c * CH) * CTX, IPC)],
                        idxa)
        for k in range(KROWS):
            pltpu.async_copy(
                inp_embed.at[idxa.at[pl.ds(k * 128, 128)]],
                rows_a.at[pl.ds(k * 128, 128)], sem)

    def wait_ctx():
        for k in range(KROWS):
            pltpu.make_async_copy(
                inp_embed.at[idxa.at[pl.ds(k * 128, 128)]],
                rows_a.at[pl.ds(k * 128, 128)], sem).wait()

    fire_ctx(0)

    def chunk_body(c, carry):
        pltpu.sync_copy(neg2d.at[pl.ds((wid * RPW + c * CH) * CTX, IPC)],
                        idxb)
        pltpu.sync_copy(out2d.at[pl.ds(wid * RPW + c * CH, CH)], idxo)
        nh = []
        for k in range(KROWS):
            nh.append(pltpu.async_copy(
                out_embed.at[idxb.at[pl.ds(k * 128, 128)]],
                rows_b.at[pl.ds(k * 128, 128)], sem2))
        oh = pltpu.async_copy(out_embed.at[idxo], out_rows, sem2)

        wait_ctx()

        def ctx_sum_body(i, cc):
            m0 = i * CTX
            a0 = rows_a[m0, pl.ds(0, 16)]
            a1 = rows_a[m0, pl.ds(16, 16)]
            for k in range(1, CTX):
                a0 = a0 + rows_a[m0 + k, pl.ds(0, 16)]
                a1 = a1 + rows_a[m0 + k, pl.ds(16, 16)]
            ctx_v[i, pl.ds(0, 16)] = a0
            ctx_v[i, pl.ds(16, 16)] = a1
            return cc
        lax.fori_loop(0, CH, ctx_sum_body, 0)

        @pl.when(c + 1 < NCHUNK)
        def _prefetch():
            fire_ctx(c + 1)

        for h in nh:
            h.wait()
        oh.wait()

        def group_body(g, cc):
            rows16 = g * 16 + iota
            negrows = [rows16 * NEG + j for j in range(NEG)]

            def d_body(dcol, acc):
                col = jnp.full((16,), dcol, jnp.int32)
                cd = plsc.load_gather(ctx_v, [rows16, col])
                od = plsc.load_gather(out_rows, [rows16, col])
                pos = acc[0] + cd * od
                new = tuple(
                    acc[1 + j] + plsc.load_gather(rows_b, [negrows[j], col]) * cd
                    for j in range(NEG))
                return (pos,) + new

            init = tuple(jnp.zeros((16,), jnp.float32) for _ in range(NEG + 1))
            res = lax.fori_loop(0, D, d_body, init)
            off = c * CH + g * 16
            pos_v[pl.ds(off, 16)] = res[0]
            for j in range(NEG):
                negT_v[j, pl.ds(off, 16)] = res[1 + j]
            return cc
        lax.fori_loop(0, CH // 16, group_body, 0)
        return carry

    lax.fori_loop(0, NCHUNK, chunk_body, 0)
    pltpu.sync_copy(pos_v, pos_hbm.at[pl.ds(wid * RPW, RPW)])
    pltpu.sync_copy(negT_v, negT_hbm.at[:, pl.ds(wid * RPW, RPW)])


_sc_call = pl.kernel(
    _sc_body,
    out_type=[jax.ShapeDtypeStruct((B,), jnp.float32),
              jax.ShapeDtypeStruct((NEG, B), jnp.float32)],
    mesh=plsc.VectorSubcoreMesh(core_axis_name="c", subcore_axis_name="s"),
    compiler_params=pltpu.CompilerParams(needs_layout_passes=False,
                                         use_tc_tiling_on_sc=False),
    scratch_types=[
        pltpu.VMEM((IPC,), jnp.int32),
        pltpu.VMEM((IPC,), jnp.int32),
        pltpu.VMEM((CH,), jnp.int32),
        pltpu.VMEM((IPC, D), jnp.float32),
        pltpu.VMEM((IPC, D), jnp.float32),
        pltpu.VMEM((CH, D), jnp.float32),
        pltpu.VMEM((CH, D), jnp.float32),
        pltpu.VMEM((RPW,), jnp.float32),
        pltpu.VMEM((NEG, RPW), jnp.float32),
        pltpu.SemaphoreType.DMA,
        pltpu.SemaphoreType.DMA,
    ],
)


def _log_sigmoid(x):
    return jnp.minimum(x, 0.0) - jnp.log(1.0 + jnp.exp(-jnp.abs(x)))


def _tc_body(pos_ref, neg_ref, o_ref):
    pos = pos_ref[...] * (1.0 / CTX)
    neg = neg_ref[...] * (1.0 / CTX)
    t1 = jnp.mean(_log_sigmoid(pos))
    t2 = jnp.sum(_log_sigmoid(-neg)) * (1.0 / B)
    o_ref[0, 0] = -(t1 + t2)


_tc_call = pl.pallas_call(
    _tc_body,
    out_shape=jax.ShapeDtypeStruct((1, 1), jnp.float32),
    out_specs=pl.BlockSpec(memory_space=pltpu.SMEM),
)


def kernel(inp_embed, out_embed, inp, out, neg):
    inp_i = inp.astype(jnp.int32).reshape(B * CTX)
    out_i = out.astype(jnp.int32).reshape(B)
    neg_i = neg.astype(jnp.int32).reshape(B * NEG)
    pad = ((0, 0), (0, 64))
    tail_i = jnp.pad(lax.slice(inp_embed.T, (0, NTILES * 128), (D, V)), pad)
    tail_o = jnp.pad(lax.slice(out_embed.T, (0, NTILES * 128), (D, V)), pad)
    lin_i, lin_o = _relay_call(inp_embed.T, out_embed.T, tail_i, tail_o)
    pos, negT = _sc_call(lin_i.reshape(V, D), lin_o.reshape(V, D),
                         inp_i, out_i, neg_i)
    loss = _tc_call(pos.reshape(128, 128), negT.reshape(NEG * B // 128, 128))
    return loss[0, 0]
